# baseline (speedup 1.0000x reference)
import jax
import jax.numpy as jnp
from jax import lax
from jax.experimental import pallas as pl
from jax.experimental.pallas import tpu as pltpu
from jax.experimental.pallas import tpu_sc as plsc
from jax._src.pallas import mpmd as pl_mpmd

C = 100000
S = 4
D = 64
B = 16384
G = C * S
P = G // 2
MOMENTUM = 0.99

NC = 2
NS = 16
NW = NC * NS
CHUNK = 128

B_PER_W = B // NW
N_CHUNKS = B_PER_W // CHUNK
IDX_ROWS = B // CHUNK

_MESH = dict(core_axis_name="c", subcore_axis_name="s")
_SC_LINEAR = pltpu.CompilerParams(use_tc_tiling_on_sc=False)


def _wid():
    return lax.axis_index("s") * NC + lax.axis_index("c")


CB = 8192
NCB = -(-C // CB)


def _tin_body(pt, out):
    y = pt[...].reshape(2 * D * 2, CB)
    ta = jnp.swapaxes(y[0:2 * D, :], 0, 1)
    tb = jnp.swapaxes(y[2 * D:, :], 0, 1)
    out[...] = jnp.stack([ta, tb], axis=0)


_t_in = pl.pallas_call(
    _tin_body,
    grid=(NCB,),
    in_specs=[pl.BlockSpec((S, D, CB), lambda k: (0, 0, k))],
    out_specs=pl.BlockSpec((2, CB, 2 * D), lambda k: (0, k, 0)),
    out_shape=jax.ShapeDtypeStruct((2, C, 2 * D), jnp.float32),
)


def _tout_body(pt, out):
    x = pt[...]
    ya = jnp.swapaxes(x[0], 0, 1)
    yb = jnp.swapaxes(x[1], 0, 1)
    out[...] = jnp.concatenate([ya, yb], axis=0).reshape(S, D, CB)


_t_out = pl.pallas_call(
    _tout_body,
    grid=(NCB,),
    in_specs=[pl.BlockSpec((2, CB, 2 * D), lambda k: (0, k, 0))],
    out_specs=pl.BlockSpec((S, D, CB), lambda k: (0, 0, k)),
    out_shape=jax.ShapeDtypeStruct((S, D, C), jnp.float32),
)


def _gr_body(protos_hbm, pidx_hbm, rows_out, idx_v, rows_v, sem):
    wid = _wid()
    pltpu.sync_copy(pidx_hbm, idx_v)
    descs = []
    for j in range(N_CHUNKS):
        descs.append(pltpu.async_copy(
            protos_hbm.at[idx_v.at[wid * N_CHUNKS + j]],
            rows_v.at[pl.ds(j * CHUNK, CHUNK), :], sem))
    for d in descs:
        d.wait()
    pltpu.sync_copy(rows_v, rows_out.at[pl.ds(wid * B_PER_W, B_PER_W), :])


_k_gr = pl.kernel(
    _gr_body,
    out_type=jax.ShapeDtypeStruct((B, 2 * D), jnp.float32),
    mesh=plsc.VectorSubcoreMesh(**_MESH),
    scratch_types=[
        pltpu.VMEM((IDX_ROWS, CHUNK), jnp.int32),
        pltpu.VMEM((B_PER_W, 2 * D), jnp.float32),
        pltpu.SemaphoreType.DMA,
    ],
)


def _gc_body(counts_hbm, idx2d_hbm, cnts_out, idx_v, cnts_v, sem):
    wid = _wid()
    pltpu.sync_copy(idx2d_hbm.at[pl.ds(wid * N_CHUNKS, N_CHUNKS), :], idx_v)
    descs = []
    for j in range(N_CHUNKS):
        descs.append(pltpu.async_copy(
            counts_hbm.at[idx_v.at[j]], cnts_v.at[j], sem))
    for d in descs:
        d.wait()
    pltpu.sync_copy(cnts_v, cnts_out.at[pl.ds(wid * N_CHUNKS, N_CHUNKS), :])


_k_gc = pl.kernel(
    _gc_body,
    out_type=jax.ShapeDtypeStruct((IDX_ROWS, CHUNK), jnp.float32),
    mesh=plsc.VectorSubcoreMesh(**_MESH),
    compiler_params=_SC_LINEAR,
    scratch_types=[
        pltpu.VMEM((N_CHUNKS, CHUNK), jnp.int32),
        pltpu.VMEM((N_CHUNKS, CHUNK), jnp.float32),
        pltpu.SemaphoreType.DMA,
    ],
)


CNT_PER_TILE = G // NS
NT_CHUNKS = (B // NS) // CHUNK
ZITER = CNT_PER_TILE // 16


def _cnt_body(ones_hbm, idx2d_hbm, sib2d_hbm, own_out, sib_out,
              idx_v, sib_v, ones_v, own_v, sib_cv, zbuf, spmem):
    wid = _wid()
    sid = lax.axis_index("s")

    def _z(i, _):
        zbuf[pl.ds(i * 16, 16)] = jnp.zeros((16,), jnp.float32)
        return _
    lax.fori_loop(0, ZITER, _z, 0)
    c0 = sid * CNT_PER_TILE
    pltpu.sync_copy(zbuf, spmem.at[pl.ds(c0, CNT_PER_TILE)])
    plsc.subcore_barrier()

    pltpu.sync_copy(ones_hbm, ones_v)
    pltpu.sync_copy(idx2d_hbm.at[pl.ds(sid * NT_CHUNKS, NT_CHUNKS), :], idx_v)
    for k in range(NT_CHUNKS):
        pltpu.sync_copy(ones_v, spmem.at[idx_v.at[k]], add=True)
    plsc.subcore_barrier()

    pltpu.sync_copy(idx2d_hbm.at[pl.ds(wid * N_CHUNKS, N_CHUNKS), :],
                    idx_v.at[pl.ds(0, N_CHUNKS), :])
    pltpu.sync_copy(sib2d_hbm.at[pl.ds(wid * N_CHUNKS, N_CHUNKS), :], sib_v)
    for k in range(N_CHUNKS):
        pltpu.sync_copy(spmem.at[idx_v.at[k]], own_v.at[k])
        pltpu.sync_copy(spmem.at[sib_v.at[k]], sib_cv.at[k])
    pltpu.sync_copy(own_v, own_out.at[pl.ds(wid * N_CHUNKS, N_CHUNKS), :])
    pltpu.sync_copy(sib_cv, sib_out.at[pl.ds(wid * N_CHUNKS, N_CHUNKS), :])


_k_cnt = pl.kernel(
    _cnt_body,
    out_type=(
        jax.ShapeDtypeStruct((IDX_ROWS, CHUNK), jnp.float32),
        jax.ShapeDtypeStruct((IDX_ROWS, CHUNK), jnp.float32),
    ),
    mesh=plsc.VectorSubcoreMesh(**_MESH),
    compiler_params=_SC_LINEAR,
    scratch_types=[
        pltpu.VMEM((NT_CHUNKS, CHUNK), jnp.int32),
        pltpu.VMEM((N_CHUNKS, CHUNK), jnp.int32),
        pltpu.VMEM((CHUNK,), jnp.float32),
        pltpu.VMEM((N_CHUNKS, CHUNK), jnp.float32),
        pltpu.VMEM((N_CHUNKS, CHUNK), jnp.float32),
        pltpu.VMEM((CNT_PER_TILE,), jnp.float32),
        pltpu.VMEM_SHARED((G,), jnp.float32),
    ],
)


BLK_I = 1024
BLK_J = 16384
NI = B // BLK_I
NJ = B // BLK_J
N_RHS = 128


def _mm_body(pid_col, pid_row, par_j, par_i, feats, prows, pcnts,
             ocnt, scnt, newrow, newcnt, acc, rhs_all):
    i = pl.program_id(0)
    j = pl.program_id(1)

    @pl.when(j == 0)
    def _init():
        acc[...] = jnp.zeros_like(acc)

    @pl.when(i == 0)
    def _build_rhs():
        par = par_j[...]
        f = feats[...]
        fe = (f * (1.0 - par)).astype(jnp.bfloat16)
        fo = (f * par).astype(jnp.bfloat16)
        rhs_all[j] = jnp.concatenate([fe, fo], axis=1)

    pm = (pid_col[...] == pid_row[...]).astype(jnp.bfloat16)
    acc[...] += jnp.dot(pm, rhs_all[j], preferred_element_type=jnp.float32)

    @pl.when(j == NJ - 1)
    def _finalize():
        a = acc[...]
        se, so = a[:, 0:D], a[:, D:2 * D]
        p = par_i[...]
        own_sum = jnp.where(p > 0.5, so, se)
        sib_sum = jnp.where(p > 0.5, se, so)
        own_cnt = ocnt[...]
        sib_cnt = scnt[...]
        pr = prows[...]
        own_pr = jnp.where(p > 0.5, pr[:, D:], pr[:, :D])
        sib_pr = jnp.where(p > 0.5, pr[:, :D], pr[:, D:])
        new_own = MOMENTUM * own_pr + (1.0 - MOMENTUM) * (own_sum / own_cnt)
        new_sib = jnp.where(
            sib_cnt > 0.5,
            MOMENTUM * sib_pr
            + (1.0 - MOMENTUM) * (sib_sum / jnp.maximum(sib_cnt, 1.0)),
            sib_pr)
        even_half = jnp.where(p > 0.5, new_sib, new_own)
        odd_half = jnp.where(p > 0.5, new_own, new_sib)
        newrow[...] = jnp.concatenate([even_half, odd_half], axis=1)
        newcnt[...] = pcnts[...] + own_cnt


_k_mm = pl.pallas_call(
    _mm_body,
    grid=(NI, NJ),
    in_specs=[
        pl.BlockSpec((BLK_I, 1), lambda i, j: (i, 0)),
        pl.BlockSpec((1, BLK_J), lambda i, j: (0, j)),
        pl.BlockSpec((BLK_J, 1), lambda i, j: (j, 0)),
        pl.BlockSpec((BLK_I, 1), lambda i, j: (i, 0)),
        pl.BlockSpec((BLK_J, D), lambda i, j: (j, 0)),
        pl.BlockSpec((BLK_I, 2 * D), lambda i, j: (i, 0)),
        pl.BlockSpec((BLK_I, 1), lambda i, j: (i, 0)),
        pl.BlockSpec((BLK_I, 1), lambda i, j: (i, 0)),
        pl.BlockSpec((BLK_I, 1), lambda i, j: (i, 0)),
    ],
    out_specs=[
        pl.BlockSpec((BLK_I, 2 * D), lambda i, j: (i, 0)),
        pl.BlockSpec((BLK_I, 1), lambda i, j: (i, 0)),
    ],
    out_shape=[
        jax.ShapeDtypeStruct((B, 2 * D), jnp.float32),
        jax.ShapeDtypeStruct((B, 1), jnp.float32),
    ],
    scratch_shapes=[
        pltpu.VMEM((BLK_I, N_RHS), jnp.float32),
        pltpu.VMEM((NJ, BLK_J, N_RHS), jnp.bfloat16),
    ],
    compiler_params=pltpu.CompilerParams(
        dimension_semantics=("arbitrary", "arbitrary")),
)


def _sr_body(newrows_hbm, pidx_hbm, protos_io, protos_out, idx_v, rows_v, sem):
    del protos_io
    wid = _wid()
    pltpu.sync_copy(pidx_hbm, idx_v)
    pltpu.sync_copy(newrows_hbm.at[pl.ds(wid * B_PER_W, B_PER_W), :], rows_v)
    descs = []
    for j in range(N_CHUNKS):
        descs.append(pltpu.async_copy(
            rows_v.at[pl.ds(j * CHUNK, CHUNK), :],
            protos_out.at[idx_v.at[wid * N_CHUNKS + j]], sem))
    for d in descs:
        d.wait()


_k_sr = pl_mpmd._mpmd_map(
    [(plsc.VectorSubcoreMesh(**_MESH), _sr_body)],
    out_types=jax.ShapeDtypeStruct((P, 2 * D), jnp.float32),
    input_output_aliases={2: 0},
    scratch_types=[
        pltpu.VMEM((IDX_ROWS, CHUNK), jnp.int32),
        pltpu.VMEM((B_PER_W, 2 * D), jnp.float32),
        pltpu.SemaphoreType.DMA,
    ],
)


def _sc_body(newcnts_hbm, idx2d_hbm, counts_io, counts_out, idx_v, cnts_v, sem):
    del counts_io
    wid = _wid()
    pltpu.sync_copy(idx2d_hbm.at[pl.ds(wid * N_CHUNKS, N_CHUNKS), :], idx_v)
    pltpu.sync_copy(newcnts_hbm.at[pl.ds(wid * N_CHUNKS, N_CHUNKS), :], cnts_v)
    descs = []
    for j in range(N_CHUNKS):
        descs.append(pltpu.async_copy(
            cnts_v.at[j], counts_out.at[idx_v.at[j]], sem))
    for d in descs:
        d.wait()


_k_sc = pl_mpmd._mpmd_map(
    [(plsc.VectorSubcoreMesh(**_MESH), _sc_body)],
    out_types=jax.ShapeDtypeStruct((G,), jnp.float32),
    input_output_aliases={2: 0},
    compiler_params=_SC_LINEAR,
    scratch_types=[
        pltpu.VMEM((N_CHUNKS, CHUNK), jnp.int32),
        pltpu.VMEM((N_CHUNKS, CHUNK), jnp.float32),
        pltpu.SemaphoreType.DMA,
    ],
)


def kernel(features, class_ids, stage_ids, prototypes, counts):
    cls = class_ids.astype(jnp.int32)
    stg = stage_ids.astype(jnp.int32)
    pair_id = cls + C * (stg // 2)
    parity = stg - 2 * (stg // 2)
    cidx = stg * C + cls
    cidx2d = cidx.reshape(IDX_ROWS, CHUNK)
    sibidx2d = ((stg ^ 1) * C + cls).reshape(IDX_ROWS, CHUNK)
    pidx2d = pair_id.reshape(IDX_ROWS, CHUNK)
    pid_f = pair_id.astype(jnp.float32)
    par_f = parity.astype(jnp.float32)

    pt = jnp.transpose(prototypes, (1, 2, 0))
    counts_lin = jnp.transpose(counts, (1, 0)).reshape(G)

    pairs = _t_in(pt).reshape(P, 2 * D)
    prows = _k_gr(pairs, pidx2d)
    pcnts = _k_gc(counts_lin, cidx2d)
    ocnt, scnt = _k_cnt(jnp.ones((CHUNK,), jnp.float32), cidx2d, sibidx2d)
    newrows, newcnts = _k_mm(
        pid_f.reshape(B, 1), pid_f.reshape(1, B),
        par_f.reshape(B, 1), par_f.reshape(B, 1),
        features, prows, pcnts.reshape(B, 1),
        ocnt.reshape(B, 1), scnt.reshape(B, 1))
    pairs_upd = _k_sr(newrows, pidx2d, pairs)
    counts_upd = _k_sc(newcnts.reshape(IDX_ROWS, CHUNK), cidx2d, counts_lin)

    protos_out = jnp.transpose(_t_out(pairs_upd.reshape(2, C, 2 * D)),
                               (2, 0, 1))
    counts_out = jnp.transpose(counts_upd.reshape(S, C), (1, 0))
    return (protos_out, counts_out)


# speedup vs baseline: 1.0195x; 1.0195x over previous
import jax
import jax.numpy as jnp
from jax import lax
from jax.experimental import pallas as pl
from jax.experimental.pallas import tpu as pltpu
from jax.experimental.pallas import tpu_sc as plsc
from jax._src.pallas import mpmd as pl_mpmd

C = 100000
S = 4
D = 64
B = 16384
G = C * S
P = G // 2
MOMENTUM = 0.99

NC = 2
NS = 16
NW = NC * NS
CHUNK = 128

B_PER_W = B // NW
N_CHUNKS = B_PER_W // CHUNK
IDX_ROWS = B // CHUNK

_MESH = dict(core_axis_name="c", subcore_axis_name="s")
_SC_LINEAR = pltpu.CompilerParams(use_tc_tiling_on_sc=False)


def _wid():
    return lax.axis_index("s") * NC + lax.axis_index("c")


CB = 8192
NCB = -(-C // CB)


def _tin_body(pt, out):
    y = pt[...].reshape(2 * D * 2, CB)
    ta = jnp.swapaxes(y[0:2 * D, :], 0, 1)
    tb = jnp.swapaxes(y[2 * D:, :], 0, 1)
    out[...] = jnp.stack([ta, tb], axis=0)


_t_in = pl.pallas_call(
    _tin_body,
    grid=(NCB,),
    in_specs=[pl.BlockSpec((S, D, CB), lambda k: (0, 0, k))],
    out_specs=pl.BlockSpec((2, CB, 2 * D), lambda k: (0, k, 0)),
    out_shape=jax.ShapeDtypeStruct((2, C, 2 * D), jnp.float32),
)


def _tout_body(pt, out):
    x = pt[...]
    ya = jnp.swapaxes(x[0], 0, 1)
    yb = jnp.swapaxes(x[1], 0, 1)
    out[...] = jnp.concatenate([ya, yb], axis=0).reshape(S, D, CB)


_t_out = pl.pallas_call(
    _tout_body,
    grid=(NCB,),
    in_specs=[pl.BlockSpec((2, CB, 2 * D), lambda k: (0, k, 0))],
    out_specs=pl.BlockSpec((S, D, CB), lambda k: (0, 0, k)),
    out_shape=jax.ShapeDtypeStruct((S, D, C), jnp.float32),
)


def _gr_body(protos_hbm, pidx_hbm, rows_out, idx_v, rows_v, sem):
    wid = _wid()
    pltpu.sync_copy(pidx_hbm, idx_v)
    descs = []
    for j in range(N_CHUNKS):
        descs.append(pltpu.async_copy(
            protos_hbm.at[idx_v.at[wid * N_CHUNKS + j]],
            rows_v.at[pl.ds(j * CHUNK, CHUNK), :], sem))
    for d in descs:
        d.wait()
    pltpu.sync_copy(rows_v, rows_out.at[pl.ds(wid * B_PER_W, B_PER_W), :])


_k_gr = pl.kernel(
    _gr_body,
    out_type=jax.ShapeDtypeStruct((B, 2 * D), jnp.float32),
    mesh=plsc.VectorSubcoreMesh(**_MESH),
    scratch_types=[
        pltpu.VMEM((IDX_ROWS, CHUNK), jnp.int32),
        pltpu.VMEM((B_PER_W, 2 * D), jnp.float32),
        pltpu.SemaphoreType.DMA,
    ],
)


def _gc_body(counts_hbm, idx2d_hbm, cnts_out, idx_v, cnts_v, sem):
    wid = _wid()
    pltpu.sync_copy(idx2d_hbm.at[pl.ds(wid * N_CHUNKS, N_CHUNKS), :], idx_v)
    descs = []
    for j in range(N_CHUNKS):
        descs.append(pltpu.async_copy(
            counts_hbm.at[idx_v.at[j]], cnts_v.at[j], sem))
    for d in descs:
        d.wait()
    pltpu.sync_copy(cnts_v, cnts_out.at[pl.ds(wid * N_CHUNKS, N_CHUNKS), :])


_k_gc = pl.kernel(
    _gc_body,
    out_type=jax.ShapeDtypeStruct((IDX_ROWS, CHUNK), jnp.float32),
    mesh=plsc.VectorSubcoreMesh(**_MESH),
    compiler_params=_SC_LINEAR,
    scratch_types=[
        pltpu.VMEM((N_CHUNKS, CHUNK), jnp.int32),
        pltpu.VMEM((N_CHUNKS, CHUNK), jnp.float32),
        pltpu.SemaphoreType.DMA,
    ],
)


BLK_I = 1024
BLK_J = 16384
NI = B // BLK_I
NJ = B // BLK_J
N_RHS = 256


def _mm_body(pid_col, pid_row, par_j, par_i, feats, prows, pcnts,
             newrow, newcnt, acc, rhs_all):
    i = pl.program_id(0)
    j = pl.program_id(1)

    @pl.when(j == 0)
    def _init():
        acc[...] = jnp.zeros_like(acc)

    @pl.when(i == 0)
    def _build_rhs():
        par = par_j[...]
        f = feats[...]
        fe = (f * (1.0 - par)).astype(jnp.bfloat16)
        fo = (f * par).astype(jnp.bfloat16)
        ce = (1.0 - par).astype(jnp.bfloat16)
        co = par.astype(jnp.bfloat16)
        pad = jnp.zeros((BLK_J, N_RHS - 2 * D - 2), jnp.bfloat16)
        rhs_all[j] = jnp.concatenate([fe, fo, ce, co, pad], axis=1)

    pm = (pid_col[...] == pid_row[...]).astype(jnp.bfloat16)
    acc[...] += jnp.dot(pm, rhs_all[j], preferred_element_type=jnp.float32)

    @pl.when(j == NJ - 1)
    def _finalize():
        a = acc[...]
        se, so = a[:, 0:D], a[:, D:2 * D]
        ce_t = a[:, 2 * D:2 * D + 1]
        co_t = a[:, 2 * D + 1:2 * D + 2]
        p = par_i[...]
        own_sum = jnp.where(p > 0.5, so, se)
        sib_sum = jnp.where(p > 0.5, se, so)
        own_cnt = jnp.where(p > 0.5, co_t, ce_t)
        sib_cnt = jnp.where(p > 0.5, ce_t, co_t)
        pr = prows[...]
        own_pr = jnp.where(p > 0.5, pr[:, D:], pr[:, :D])
        sib_pr = jnp.where(p > 0.5, pr[:, :D], pr[:, D:])
        new_own = MOMENTUM * own_pr + (1.0 - MOMENTUM) * (own_sum / own_cnt)
        new_sib = jnp.where(
            sib_cnt > 0.5,
            MOMENTUM * sib_pr
            + (1.0 - MOMENTUM) * (sib_sum / jnp.maximum(sib_cnt, 1.0)),
            sib_pr)
        even_half = jnp.where(p > 0.5, new_sib, new_own)
        odd_half = jnp.where(p > 0.5, new_own, new_sib)
        newrow[...] = jnp.concatenate([even_half, odd_half], axis=1)
        newcnt[...] = pcnts[...] + own_cnt


_k_mm = pl.pallas_call(
    _mm_body,
    grid=(NI, NJ),
    in_specs=[
        pl.BlockSpec((BLK_I, 1), lambda i, j: (i, 0)),
        pl.BlockSpec((1, BLK_J), lambda i, j: (0, j)),
        pl.BlockSpec((BLK_J, 1), lambda i, j: (j, 0)),
        pl.BlockSpec((BLK_I, 1), lambda i, j: (i, 0)),
        pl.BlockSpec((BLK_J, D), lambda i, j: (j, 0)),
        pl.BlockSpec((BLK_I, 2 * D), lambda i, j: (i, 0)),
        pl.BlockSpec((BLK_I, 1), lambda i, j: (i, 0)),
    ],
    out_specs=[
        pl.BlockSpec((BLK_I, 2 * D), lambda i, j: (i, 0)),
        pl.BlockSpec((BLK_I, 1), lambda i, j: (i, 0)),
    ],
    out_shape=[
        jax.ShapeDtypeStruct((B, 2 * D), jnp.float32),
        jax.ShapeDtypeStruct((B, 1), jnp.float32),
    ],
    scratch_shapes=[
        pltpu.VMEM((BLK_I, N_RHS), jnp.float32),
        pltpu.VMEM((NJ, BLK_J, N_RHS), jnp.bfloat16),
    ],
    compiler_params=pltpu.CompilerParams(
        dimension_semantics=("arbitrary", "arbitrary")),
)


def _sr_body(newrows_hbm, pidx_hbm, protos_io, protos_out, idx_v, rows_v, sem):
    del protos_io
    wid = _wid()
    pltpu.sync_copy(pidx_hbm, idx_v)
    pltpu.sync_copy(newrows_hbm.at[pl.ds(wid * B_PER_W, B_PER_W), :], rows_v)
    descs = []
    for j in range(N_CHUNKS):
        descs.append(pltpu.async_copy(
            rows_v.at[pl.ds(j * CHUNK, CHUNK), :],
            protos_out.at[idx_v.at[wid * N_CHUNKS + j]], sem))
    for d in descs:
        d.wait()


_k_sr = pl_mpmd._mpmd_map(
    [(plsc.VectorSubcoreMesh(**_MESH), _sr_body)],
    out_types=jax.ShapeDtypeStruct((P, 2 * D), jnp.float32),
    input_output_aliases={2: 0},
    scratch_types=[
        pltpu.VMEM((IDX_ROWS, CHUNK), jnp.int32),
        pltpu.VMEM((B_PER_W, 2 * D), jnp.float32),
        pltpu.SemaphoreType.DMA,
    ],
)


def _sc_body(newcnts_hbm, idx2d_hbm, counts_io, counts_out, idx_v, cnts_v, sem):
    del counts_io
    wid = _wid()
    pltpu.sync_copy(idx2d_hbm.at[pl.ds(wid * N_CHUNKS, N_CHUNKS), :], idx_v)
    pltpu.sync_copy(newcnts_hbm.at[pl.ds(wid * N_CHUNKS, N_CHUNKS), :], cnts_v)
    descs = []
    for j in range(N_CHUNKS):
        descs.append(pltpu.async_copy(
            cnts_v.at[j], counts_out.at[idx_v.at[j]], sem))
    for d in descs:
        d.wait()


_k_sc = pl_mpmd._mpmd_map(
    [(plsc.VectorSubcoreMesh(**_MESH), _sc_body)],
    out_types=jax.ShapeDtypeStruct((G,), jnp.float32),
    input_output_aliases={2: 0},
    compiler_params=_SC_LINEAR,
    scratch_types=[
        pltpu.VMEM((N_CHUNKS, CHUNK), jnp.int32),
        pltpu.VMEM((N_CHUNKS, CHUNK), jnp.float32),
        pltpu.SemaphoreType.DMA,
    ],
)


def kernel(features, class_ids, stage_ids, prototypes, counts):
    cls = class_ids.astype(jnp.int32)
    stg = stage_ids.astype(jnp.int32)
    pair_id = cls + C * (stg // 2)
    parity = stg - 2 * (stg // 2)
    cidx = stg * C + cls
    cidx2d = cidx.reshape(IDX_ROWS, CHUNK)
    pidx2d = pair_id.reshape(IDX_ROWS, CHUNK)
    pid_f = pair_id.astype(jnp.float32)
    par_f = parity.astype(jnp.float32)

    pt = jnp.transpose(prototypes, (1, 2, 0))
    counts_lin = jnp.transpose(counts, (1, 0)).reshape(G)

    pairs = _t_in(pt).reshape(P, 2 * D)
    prows = _k_gr(pairs, pidx2d)
    pcnts = _k_gc(counts_lin, cidx2d)
    newrows, newcnts = _k_mm(
        pid_f.reshape(B, 1), pid_f.reshape(1, B),
        par_f.reshape(B, 1), par_f.reshape(B, 1),
        features, prows, pcnts.reshape(B, 1))
    pairs_upd = _k_sr(newrows, pidx2d, pairs)
    counts_upd = _k_sc(newcnts.reshape(IDX_ROWS, CHUNK), cidx2d, counts_lin)

    protos_out = jnp.transpose(_t_out(pairs_upd.reshape(2, C, 2 * D)),
                               (2, 0, 1))
    counts_out = jnp.transpose(counts_upd.reshape(S, C), (1, 0))
    return (protos_out, counts_out)


# speedup vs baseline: 1.0317x; 1.0119x over previous
import jax
import jax.numpy as jnp
from jax import lax
from jax.experimental import pallas as pl
from jax.experimental.pallas import tpu as pltpu
from jax.experimental.pallas import tpu_sc as plsc
from jax._src.pallas import mpmd as pl_mpmd

C = 100000
S = 4
D = 64
B = 16384
G = C * S
P = G // 2
MOMENTUM = 0.99

NC = 2
NS = 16
NW = NC * NS
CHUNK = 128

B_PER_W = B // NW
N_CHUNKS = B_PER_W // CHUNK
IDX_ROWS = B // CHUNK

_MESH = dict(core_axis_name="c", subcore_axis_name="s")
_SC_LINEAR = pltpu.CompilerParams(use_tc_tiling_on_sc=False)


def _wid():
    return lax.axis_index("s") * NC + lax.axis_index("c")


CB = 8192
NCB = -(-C // CB)


def _tin_body(pt, out):
    y = pt[...].reshape(2 * D * 2, CB)
    ta = jnp.swapaxes(y[0:2 * D, :], 0, 1)
    tb = jnp.swapaxes(y[2 * D:, :], 0, 1)
    out[...] = jnp.stack([ta, tb], axis=0)


_t_in = pl.pallas_call(
    _tin_body,
    grid=(NCB,),
    in_specs=[pl.BlockSpec((S, D, CB), lambda k: (0, 0, k))],
    out_specs=pl.BlockSpec((2, CB, 2 * D), lambda k: (0, k, 0)),
    out_shape=jax.ShapeDtypeStruct((2, C, 2 * D), jnp.float32),
)


def _tout_body(pt, out):
    x = pt[...]
    ya = jnp.swapaxes(x[0], 0, 1)
    yb = jnp.swapaxes(x[1], 0, 1)
    out[...] = jnp.concatenate([ya, yb], axis=0).reshape(S, D, CB)


_t_out = pl.pallas_call(
    _tout_body,
    grid=(NCB,),
    in_specs=[pl.BlockSpec((2, CB, 2 * D), lambda k: (0, k, 0))],
    out_specs=pl.BlockSpec((S, D, CB), lambda k: (0, 0, k)),
    out_shape=jax.ShapeDtypeStruct((S, D, C), jnp.float32),
)


def _gr_body(protos_hbm, pidx_hbm, rows_out, idx_v, rows_v, sem):
    wid = _wid()
    pltpu.sync_copy(pidx_hbm, idx_v)
    descs = []
    for j in range(N_CHUNKS):
        descs.append(pltpu.async_copy(
            protos_hbm.at[idx_v.at[wid * N_CHUNKS + j]],
            rows_v.at[pl.ds(j * CHUNK, CHUNK), :], sem))
    for d in descs:
        d.wait()
    pltpu.sync_copy(rows_v, rows_out.at[pl.ds(wid * B_PER_W, B_PER_W), :])


_k_gr = pl.kernel(
    _gr_body,
    out_type=jax.ShapeDtypeStruct((B, 2 * D), jnp.float32),
    mesh=plsc.VectorSubcoreMesh(**_MESH),
    scratch_types=[
        pltpu.VMEM((IDX_ROWS, CHUNK), jnp.int32),
        pltpu.VMEM((B_PER_W, 2 * D), jnp.float32),
        pltpu.SemaphoreType.DMA,
    ],
)


def _gc_body(counts_hbm, idx2d_hbm, cnts_out, idx_v, cnts_v, sem):
    wid = _wid()
    pltpu.sync_copy(idx2d_hbm.at[pl.ds(wid * N_CHUNKS, N_CHUNKS), :], idx_v)
    descs = []
    for j in range(N_CHUNKS):
        descs.append(pltpu.async_copy(
            counts_hbm.at[idx_v.at[j]], cnts_v.at[j], sem))
    for d in descs:
        d.wait()
    pltpu.sync_copy(cnts_v, cnts_out.at[pl.ds(wid * N_CHUNKS, N_CHUNKS), :])


_k_gc = pl.kernel(
    _gc_body,
    out_type=jax.ShapeDtypeStruct((IDX_ROWS, CHUNK), jnp.float32),
    mesh=plsc.VectorSubcoreMesh(**_MESH),
    compiler_params=_SC_LINEAR,
    scratch_types=[
        pltpu.VMEM((N_CHUNKS, CHUNK), jnp.int32),
        pltpu.VMEM((N_CHUNKS, CHUNK), jnp.float32),
        pltpu.SemaphoreType.DMA,
    ],
)


BLK_I = 1024
BLK_J = 16384
NI = B // BLK_I
NJ = B // BLK_J
N_RHS = 256


def _mm_body(cols_i, pid_row, cols_j, feats, prows,
             newrow, newcnt, acc, rhs_all):
    i = pl.program_id(0)
    j = pl.program_id(1)

    @pl.when(j == 0)
    def _init():
        acc[...] = jnp.zeros_like(acc)

    @pl.when(i == 0)
    def _build_rhs():
        par = cols_j[:, 1:2]
        f = feats[...]
        fe = (f * (1.0 - par)).astype(jnp.bfloat16)
        fo = (f * par).astype(jnp.bfloat16)
        ce = (1.0 - par).astype(jnp.bfloat16)
        co = par.astype(jnp.bfloat16)
        pad = jnp.zeros((BLK_J, N_RHS - 2 * D - 2), jnp.bfloat16)
        rhs_all[j] = jnp.concatenate([fe, fo, ce, co, pad], axis=1)

    pm = (cols_i[:, 0:1] == pid_row[...]).astype(jnp.bfloat16)
    acc[...] += jnp.dot(pm, rhs_all[j], preferred_element_type=jnp.float32)

    @pl.when(j == NJ - 1)
    def _finalize():
        a = acc[...]
        se, so = a[:, 0:D], a[:, D:2 * D]
        ce_t = a[:, 2 * D:2 * D + 1]
        co_t = a[:, 2 * D + 1:2 * D + 2]
        p = cols_i[:, 1:2]
        own_sum = jnp.where(p > 0.5, so, se)
        sib_sum = jnp.where(p > 0.5, se, so)
        own_cnt = jnp.where(p > 0.5, co_t, ce_t)
        sib_cnt = jnp.where(p > 0.5, ce_t, co_t)
        pr = prows[...]
        own_pr = jnp.where(p > 0.5, pr[:, D:], pr[:, :D])
        sib_pr = jnp.where(p > 0.5, pr[:, :D], pr[:, D:])
        new_own = MOMENTUM * own_pr + (1.0 - MOMENTUM) * (own_sum / own_cnt)
        new_sib = jnp.where(
            sib_cnt > 0.5,
            MOMENTUM * sib_pr
            + (1.0 - MOMENTUM) * (sib_sum / jnp.maximum(sib_cnt, 1.0)),
            sib_pr)
        even_half = jnp.where(p > 0.5, new_sib, new_own)
        odd_half = jnp.where(p > 0.5, new_own, new_sib)
        newrow[...] = jnp.concatenate([even_half, odd_half], axis=1)
        newcnt[...] = cols_i[:, 2:3] + own_cnt


_k_mm = pl.pallas_call(
    _mm_body,
    grid=(NI, NJ),
    in_specs=[
        pl.BlockSpec((BLK_I, 8), lambda i, j: (i, 0)),
        pl.BlockSpec((1, BLK_J), lambda i, j: (0, j)),
        pl.BlockSpec((BLK_J, 8), lambda i, j: (j, 0)),
        pl.BlockSpec((BLK_J, D), lambda i, j: (j, 0)),
        pl.BlockSpec((BLK_I, 2 * D), lambda i, j: (i, 0)),
    ],
    out_specs=[
        pl.BlockSpec((BLK_I, 2 * D), lambda i, j: (i, 0)),
        pl.BlockSpec((BLK_I, 1), lambda i, j: (i, 0)),
    ],
    out_shape=[
        jax.ShapeDtypeStruct((B, 2 * D), jnp.float32),
        jax.ShapeDtypeStruct((B, 1), jnp.float32),
    ],
    scratch_shapes=[
        pltpu.VMEM((BLK_I, N_RHS), jnp.float32),
        pltpu.VMEM((NJ, BLK_J, N_RHS), jnp.bfloat16),
    ],
    compiler_params=pltpu.CompilerParams(
        dimension_semantics=("arbitrary", "arbitrary")),
)


def _sr_body(newrows_hbm, pidx_hbm, protos_io, protos_out, idx_v, rows_v, sem):
    del protos_io
    wid = _wid()
    pltpu.sync_copy(pidx_hbm, idx_v)
    pltpu.sync_copy(newrows_hbm.at[pl.ds(wid * B_PER_W, B_PER_W), :], rows_v)
    descs = []
    for j in range(N_CHUNKS):
        descs.append(pltpu.async_copy(
            rows_v.at[pl.ds(j * CHUNK, CHUNK), :],
            protos_out.at[idx_v.at[wid * N_CHUNKS + j]], sem))
    for d in descs:
        d.wait()


_k_sr = pl_mpmd._mpmd_map(
    [(plsc.VectorSubcoreMesh(**_MESH), _sr_body)],
    out_types=jax.ShapeDtypeStruct((P, 2 * D), jnp.float32),
    input_output_aliases={2: 0},
    scratch_types=[
        pltpu.VMEM((IDX_ROWS, CHUNK), jnp.int32),
        pltpu.VMEM((B_PER_W, 2 * D), jnp.float32),
        pltpu.SemaphoreType.DMA,
    ],
)


def _sc_body(newcnts_hbm, idx2d_hbm, counts_io, counts_out, idx_v, cnts_v, sem):
    del counts_io
    wid = _wid()
    pltpu.sync_copy(idx2d_hbm.at[pl.ds(wid * N_CHUNKS, N_CHUNKS), :], idx_v)
    pltpu.sync_copy(newcnts_hbm.at[pl.ds(wid * N_CHUNKS, N_CHUNKS), :], cnts_v)
    descs = []
    for j in range(N_CHUNKS):
        descs.append(pltpu.async_copy(
            cnts_v.at[j], counts_out.at[idx_v.at[j]], sem))
    for d in descs:
        d.wait()


_k_sc = pl_mpmd._mpmd_map(
    [(plsc.VectorSubcoreMesh(**_MESH), _sc_body)],
    out_types=jax.ShapeDtypeStruct((G,), jnp.float32),
    input_output_aliases={2: 0},
    compiler_params=_SC_LINEAR,
    scratch_types=[
        pltpu.VMEM((N_CHUNKS, CHUNK), jnp.int32),
        pltpu.VMEM((N_CHUNKS, CHUNK), jnp.float32),
        pltpu.SemaphoreType.DMA,
    ],
)


def kernel(features, class_ids, stage_ids, prototypes, counts):
    cls = class_ids.astype(jnp.int32)
    stg = stage_ids.astype(jnp.int32)
    pair_id = cls + C * (stg // 2)
    parity = stg - 2 * (stg // 2)
    cidx = stg * C + cls
    cidx2d = cidx.reshape(IDX_ROWS, CHUNK)
    pidx2d = pair_id.reshape(IDX_ROWS, CHUNK)
    pid_f = pair_id.astype(jnp.float32)
    par_f = parity.astype(jnp.float32)

    pt = jnp.transpose(prototypes, (1, 2, 0))
    counts_lin = jnp.transpose(counts, (1, 0)).reshape(G)

    pairs = _t_in(pt).reshape(P, 2 * D)
    prows = _k_gr(pairs, pidx2d)
    pcnts = _k_gc(counts_lin, cidx2d)
    cols = jnp.concatenate(
        [pid_f.reshape(B, 1), par_f.reshape(B, 1), pcnts.reshape(B, 1),
         jnp.zeros((B, 5), jnp.float32)], axis=1)
    newrows, newcnts = _k_mm(
        cols, pid_f.reshape(1, B), cols, features, prows)
    pairs_upd = _k_sr(newrows, pidx2d, pairs)
    counts_upd = _k_sc(newcnts.reshape(IDX_ROWS, CHUNK), cidx2d, counts_lin)

    protos_out = jnp.transpose(_t_out(pairs_upd.reshape(2, C, 2 * D)),
                               (2, 0, 1))
    counts_out = jnp.transpose(counts_upd.reshape(S, C), (1, 0))
    return (protos_out, counts_out)
